# baseline (speedup 1.0000x reference)
import functools

import jax
import jax.numpy as jnp
from jax import lax
from jax.experimental import pallas as pl
from jax.experimental.pallas import tpu as pltpu
from jax.experimental.pallas import tpu_sc as plsc

B = 16384
D = 64
NC = 2
NS = 16
NW = NC * NS
BPW = B // NW
CHUNK = 128
NCHUNK = BPW // CHUNK
GROUPS = BPW // 16


def _body(user_emb, movie_emb, uids, mids, out_hbm,
          uidx_v, midx_v, urows_v, mrows_v, out_v, sem):
    w = lax.axis_index("c") * NS + lax.axis_index("s")

    pltpu.sync_copy(uids.at[pl.ds(w * NCHUNK, NCHUNK)], uidx_v)
    pltpu.sync_copy(mids.at[pl.ds(w * NCHUNK, NCHUNK)], midx_v)

    copies = []
    for c in range(NCHUNK):
        copies.append(pltpu.async_copy(
            user_emb.at[uidx_v.at[c]],
            urows_v.at[pl.ds(c * CHUNK, CHUNK)], sem))
        copies.append(pltpu.async_copy(
            movie_emb.at[midx_v.at[c]],
            mrows_v.at[pl.ds(c * CHUNK, CHUNK)], sem))
    for cp in copies:
        cp.wait()

    lane = jax.lax.iota(jnp.int32, 16)

    def group(g, carry):
        base = g * 16
        res = jnp.zeros((16,), jnp.float32)
        for e in range(16):
            j = base + e
            acc = jnp.zeros((16,), jnp.float32)
            for k in range(D // 16):
                u = urows_v[j, pl.ds(k * 16, 16)]
                m = mrows_v[j, pl.ds(k * 16, 16)]
                acc = acc + u * m
            res = jnp.where(lane == e, jnp.sum(acc), res)
        out_v[pl.ds(base, 16)] = res
        return carry

    lax.fori_loop(0, GROUPS, group, 0)

    pltpu.sync_copy(out_v, out_hbm.at[pl.ds(w * BPW, BPW)])


@jax.jit
def _mf_kernel(user_ids, movie_ids, user_emb, movie_emb):
    uids = user_ids.astype(jnp.int32).reshape(NW * NCHUNK, CHUNK)
    mids = movie_ids.astype(jnp.int32).reshape(NW * NCHUNK, CHUNK)

    mesh = plsc.VectorSubcoreMesh(core_axis_name="c", subcore_axis_name="s")
    run = functools.partial(
        pl.kernel,
        mesh=mesh,
        compiler_params=pltpu.CompilerParams(
            needs_layout_passes=False, use_tc_tiling_on_sc=False),
        out_type=jax.ShapeDtypeStruct((B,), jnp.float32),
        scratch_types=[
            pltpu.VMEM((NCHUNK, CHUNK), jnp.int32),
            pltpu.VMEM((NCHUNK, CHUNK), jnp.int32),
            pltpu.VMEM((BPW, D), jnp.float32),
            pltpu.VMEM((BPW, D), jnp.float32),
            pltpu.VMEM((BPW,), jnp.float32),
            pltpu.SemaphoreType.DMA,
        ],
    )(_body)
    return run(user_emb, movie_emb, uids, mids)


def kernel(user_ids, movie_ids, user_emb, movie_emb):
    return _mf_kernel(user_ids, movie_ids, user_emb, movie_emb)
